# Initial kernel scaffold; baseline (speedup 1.0000x reference)
import jax
import jax.numpy as jnp
from jax.experimental import pallas as pl


def kernel(x, edge_index, W1, b1, gamma1, beta1, W2, b2):
    raise NotImplementedError("write your pallas kernel here")


# speedup vs baseline: 15.3900x; 15.3900x over previous
import functools

import jax
import jax.numpy as jnp
from jax import lax
from jax.experimental import pallas as pl
from jax.experimental.pallas import tpu as pltpu
from jax.experimental.pallas import tpu_sc as plsc

N = 10000
F_IN = 128
H = 64
C = 40
E = 320000

NC = 2
NS = 16
NW = NC * NS
CHUNK = 128
CPW = 79
EPW = CPW * CHUNK
EPAD = NW * EPW
NPAD = 10112
RPT = NPAD // NS
D2 = 48
DW = 16

_mesh = plsc.VectorSubcoreMesh(
    core_axis_name="c", subcore_axis_name="s", num_cores=NC, num_subcores=NS)
_sc_params = pltpu.CompilerParams(use_tc_tiling_on_sc=False)


def _zero_shared_slice(zrows_v, acc_sh, base):
    for k in range(RPT // CHUNK):
        pltpu.sync_copy(zrows_v, acc_sh.at[pl.ds(base + k * CHUNK, CHUNK)])
    rem = RPT % CHUNK
    if rem:
        pltpu.sync_copy(zrows_v.at[pl.ds(0, rem)],
                        acc_sh.at[pl.ds(base + (RPT // CHUNK) * CHUNK, rem)])


@functools.partial(
    pl.kernel,
    out_type=jax.ShapeDtypeStruct((NC, NPAD, DW), jnp.float32),
    mesh=_mesh,
    compiler_params=_sc_params,
    scratch_types=[
        pltpu.VMEM((CHUNK,), jnp.int32),
        pltpu.VMEM((CHUNK, DW), jnp.float32),
        pltpu.VMEM((CHUNK, DW), jnp.float32),
        pltpu.VMEM_SHARED((NPAD, DW), jnp.float32),
    ],
)
def _deg_kernel(c_hbm, out_hbm, cidx_v, ones_v, zero_v, dacc_sh):
    cid = lax.axis_index("c")
    sid = lax.axis_index("s")
    wid = cid * NS + sid

    def fill(i, _):
        ones_v[i, :] = jnp.ones((DW,), jnp.float32)
        zero_v[i, :] = jnp.zeros((DW,), jnp.float32)
        return 0

    lax.fori_loop(0, CHUNK, fill, 0)

    base = sid * RPT
    _zero_shared_slice(zero_v, dacc_sh, base)
    plsc.subcore_barrier()

    ebase = wid * EPW

    def body(j, _):
        pltpu.sync_copy(c_hbm.at[pl.ds(ebase + j * CHUNK, CHUNK)], cidx_v)
        pltpu.sync_copy(ones_v, dacc_sh.at[cidx_v], add=True)
        return 0

    lax.fori_loop(0, CPW, body, 0)

    plsc.subcore_barrier()
    pltpu.sync_copy(dacc_sh.at[pl.ds(base, RPT)],
                    out_hbm.at[cid, pl.ds(base, RPT)])


def _make_prop(D):
    lw = D // 16

    @functools.partial(
        pl.kernel,
        out_type=jax.ShapeDtypeStruct((NC, NPAD, D), jnp.float32),
        mesh=_mesh,
        compiler_params=_sc_params,
        scratch_types=[
            pltpu.VMEM((CHUNK,), jnp.int32),
            pltpu.VMEM((CHUNK,), jnp.int32),
            pltpu.VMEM((CHUNK, D), jnp.float32),
            pltpu.VMEM_SHARED((NPAD, D), jnp.float32),
        ],
    )
    def _prop(r_hbm, c_hbm, hs_hbm, out_hbm, ridx_v, cidx_v, rows_v, acc_sh):
        cid = lax.axis_index("c")
        sid = lax.axis_index("s")
        wid = cid * NS + sid

        def zfill(i, _):
            rows_v[i // lw, pl.ds((i % lw) * 16, 16)] = jnp.zeros(
                (16,), jnp.float32)
            return 0

        lax.fori_loop(0, CHUNK * lw, zfill, 0)

        base = sid * RPT
        _zero_shared_slice(rows_v, acc_sh, base)
        plsc.subcore_barrier()

        ebase = wid * EPW

        def body(j, _):
            off = ebase + j * CHUNK
            pltpu.sync_copy(r_hbm.at[pl.ds(off, CHUNK)], ridx_v)
            pltpu.sync_copy(c_hbm.at[pl.ds(off, CHUNK)], cidx_v)
            pltpu.sync_copy(hs_hbm.at[ridx_v], rows_v)
            pltpu.sync_copy(rows_v, acc_sh.at[cidx_v], add=True)
            return 0

        lax.fori_loop(0, CPW, body, 0)

        plsc.subcore_barrier()
        pltpu.sync_copy(acc_sh.at[pl.ds(base, RPT)],
                        out_hbm.at[cid, pl.ds(base, RPT)])

    return _prop


_prop1 = _make_prop(H)
_prop2 = _make_prop(D2)


def _dinv_from(deg_ref):
    deg = deg_ref[0, :, 0:1] + deg_ref[1, :, 0:1] + 1.0
    return lax.rsqrt(deg)


def _row_mask():
    return lax.broadcasted_iota(jnp.int32, (NPAD, 1), 0) < N


def _tc1(x_ref, w1_ref, deg_ref, hs1_ref):
    dinv = _dinv_from(deg_ref)
    h = jnp.dot(x_ref[...], w1_ref[...], preferred_element_type=jnp.float32)
    hs1_ref[...] = jnp.where(_row_mask(), h * dinv, 0.0)


def _tc2(acc_ref, hs1_ref, deg_ref, g_ref, be_ref, b1_ref, w2_ref, hs2_ref):
    dinv = _dinv_from(deg_ref)
    mask = _row_mask()
    s = (acc_ref[0] + acc_ref[1] + hs1_ref[...]) * dinv + b1_ref[...]
    s = jnp.where(mask, s, 0.0)
    inv_n = jnp.float32(1.0 / N)
    mu = jnp.sum(s, axis=0, keepdims=True) * inv_n
    var = jnp.sum(s * s, axis=0, keepdims=True) * inv_n - mu * mu
    g = g_ref[...] * (s - mu) * lax.rsqrt(var + 1e-5) + be_ref[...]
    g = jnp.where(mask, jnp.maximum(g, 0.0), 0.0)
    h2 = jnp.dot(g, w2_ref[...], preferred_element_type=jnp.float32)
    hs2_ref[...] = h2 * dinv


def _tc3(acc_ref, hs2_ref, deg_ref, b2_ref, out_ref):
    dinv = _dinv_from(deg_ref)
    o = (acc_ref[0] + acc_ref[1] + hs2_ref[...]) * dinv + b2_ref[...]
    out_ref[...] = o[:N, :C]


def kernel(x, edge_index, W1, b1, gamma1, beta1, W2, b2):
    pad = jnp.full((EPAD - E,), N, dtype=jnp.int32)
    r_p = jnp.concatenate([edge_index[0], pad])
    c_p = jnp.concatenate([edge_index[1], pad])
    x_p = jnp.pad(x, ((0, NPAD - N), (0, 0)))
    w2_p = jnp.pad(W2, ((0, 0), (0, D2 - C)))
    b1r = b1.reshape(1, H)
    b2r = jnp.pad(b2, (0, D2 - C)).reshape(1, D2)
    g1r = gamma1.reshape(1, H)
    be1r = beta1.reshape(1, H)

    deg = _deg_kernel(c_p)

    hs1 = pl.pallas_call(
        _tc1,
        out_shape=jax.ShapeDtypeStruct((NPAD, H), jnp.float32),
    )(x_p, W1, deg)

    acc1 = _prop1(r_p, c_p, hs1)

    hs2 = pl.pallas_call(
        _tc2,
        out_shape=jax.ShapeDtypeStruct((NPAD, D2), jnp.float32),
    )(acc1, hs1, deg, g1r, be1r, b1r, w2_p)

    acc2 = _prop2(r_p, c_p, hs2)

    out = pl.pallas_call(
        _tc3,
        out_shape=jax.ShapeDtypeStruct((N, C), jnp.float32),
    )(acc2, hs2, deg, b2r)

    return out


# speedup vs baseline: 19.1541x; 1.2446x over previous
import functools

import jax
import jax.numpy as jnp
from jax import lax
from jax.experimental import pallas as pl
from jax.experimental.pallas import tpu as pltpu
from jax.experimental.pallas import tpu_sc as plsc

N = 10000
F_IN = 128
H = 64
C = 40
E = 320000

NC = 2
NS = 16
NW = NC * NS
CHUNK = 128
CPW = 80
EPW = CPW * CHUNK
EPAD = NW * EPW
NPAD = 10112
RPT = NPAD // NS
D2 = 48
DW = 16

_mesh = plsc.VectorSubcoreMesh(
    core_axis_name="c", subcore_axis_name="s", num_cores=NC, num_subcores=NS)
_sc_params = pltpu.CompilerParams(use_tc_tiling_on_sc=False)


def _zero_shared_slice(zrows_v, acc_sh, base):
    for k in range(RPT // CHUNK):
        pltpu.sync_copy(zrows_v, acc_sh.at[pl.ds(base + k * CHUNK, CHUNK)])
    rem = RPT % CHUNK
    if rem:
        pltpu.sync_copy(zrows_v.at[pl.ds(0, rem)],
                        acc_sh.at[pl.ds(base + (RPT // CHUNK) * CHUNK, rem)])


@functools.partial(
    pl.kernel,
    out_type=jax.ShapeDtypeStruct((NC, NPAD, DW), jnp.float32),
    mesh=_mesh,
    compiler_params=_sc_params,
    scratch_types=[
        pltpu.VMEM((CPW, CHUNK), jnp.int32),
        pltpu.VMEM((CHUNK, DW), jnp.float32),
        pltpu.VMEM((CHUNK, DW), jnp.float32),
        pltpu.VMEM_SHARED((NPAD, DW), jnp.float32),
    ],
)
def _deg_kernel(c3_hbm, out_hbm, cidx_v, ones_v, zero_v, dacc_sh):
    cid = lax.axis_index("c")
    sid = lax.axis_index("s")
    wid = cid * NS + sid

    pltpu.sync_copy(c3_hbm.at[wid], cidx_v)

    def fill(i, _):
        ones_v[i, :] = jnp.ones((DW,), jnp.float32)
        zero_v[i, :] = jnp.zeros((DW,), jnp.float32)
        return 0

    lax.fori_loop(0, CHUNK, fill, 0)

    base = sid * RPT
    _zero_shared_slice(zero_v, dacc_sh, base)
    plsc.subcore_barrier()

    def body(j, _):
        pltpu.sync_copy(ones_v, dacc_sh.at[cidx_v.at[j]], add=True)
        return 0

    lax.fori_loop(0, CPW, body, 0)

    plsc.subcore_barrier()
    pltpu.sync_copy(dacc_sh.at[pl.ds(base, RPT)],
                    out_hbm.at[cid, pl.ds(base, RPT)])


def _make_prop(D):
    lw = D // 16

    @functools.partial(
        pl.kernel,
        out_type=jax.ShapeDtypeStruct((NC, NPAD, D), jnp.float32),
        mesh=_mesh,
        compiler_params=_sc_params,
        scratch_types=[
            pltpu.VMEM((CPW, CHUNK), jnp.int32),
            pltpu.VMEM((CPW, CHUNK), jnp.int32),
            pltpu.VMEM((CHUNK, D), jnp.float32),
            pltpu.VMEM((CHUNK, D), jnp.float32),
            pltpu.VMEM_SHARED((NPAD, D), jnp.float32),
            pltpu.SemaphoreType.DMA,
            pltpu.SemaphoreType.DMA,
        ],
    )
    def _prop(r3_hbm, c3_hbm, hs_hbm, out_hbm, ridx_v, cidx_v, rows0_v,
              rows1_v, acc_sh, sem0, sem1):
        cid = lax.axis_index("c")
        sid = lax.axis_index("s")
        wid = cid * NS + sid

        pltpu.sync_copy(r3_hbm.at[wid], ridx_v)
        pltpu.sync_copy(c3_hbm.at[wid], cidx_v)

        def zfill(i, _):
            rows0_v[i // lw, pl.ds((i % lw) * 16, 16)] = jnp.zeros(
                (16,), jnp.float32)
            return 0

        lax.fori_loop(0, CHUNK * lw, zfill, 0)

        base = sid * RPT
        _zero_shared_slice(rows0_v, acc_sh, base)
        plsc.subcore_barrier()

        pltpu.async_copy(hs_hbm.at[ridx_v.at[0]], rows0_v, sem0)

        def body(i, _):
            j0 = 2 * i
            pltpu.async_copy(hs_hbm.at[ridx_v.at[j0 + 1]], rows1_v, sem1)
            pltpu.make_async_copy(hs_hbm.at[ridx_v.at[j0]], rows0_v,
                                  sem0).wait()
            pltpu.sync_copy(rows0_v, acc_sh.at[cidx_v.at[j0]], add=True)

            @pl.when(j0 + 2 < CPW)
            def _():
                pltpu.async_copy(hs_hbm.at[ridx_v.at[j0 + 2]], rows0_v, sem0)

            pltpu.make_async_copy(hs_hbm.at[ridx_v.at[j0 + 1]], rows1_v,
                                  sem1).wait()
            pltpu.sync_copy(rows1_v, acc_sh.at[cidx_v.at[j0 + 1]], add=True)
            return 0

        lax.fori_loop(0, CPW // 2, body, 0)

        plsc.subcore_barrier()
        pltpu.sync_copy(acc_sh.at[pl.ds(base, RPT)],
                        out_hbm.at[cid, pl.ds(base, RPT)])

    return _prop


_prop1 = _make_prop(H)
_prop2 = _make_prop(D2)


def _dinv_from(deg_ref):
    deg = deg_ref[0, :, 0:1] + deg_ref[1, :, 0:1] + 1.0
    return lax.rsqrt(deg)


def _row_mask():
    return lax.broadcasted_iota(jnp.int32, (NPAD, 1), 0) < N


def _tc1(x_ref, w1_ref, deg_ref, hs1_ref):
    dinv = _dinv_from(deg_ref)
    h = jnp.dot(x_ref[...], w1_ref[...], preferred_element_type=jnp.float32)
    hs1_ref[...] = jnp.where(_row_mask(), h * dinv, 0.0)


def _tc2(acc_ref, hs1_ref, deg_ref, g_ref, be_ref, b1_ref, w2_ref, hs2_ref):
    dinv = _dinv_from(deg_ref)
    mask = _row_mask()
    s = (acc_ref[0] + acc_ref[1] + hs1_ref[...]) * dinv + b1_ref[...]
    s = jnp.where(mask, s, 0.0)
    inv_n = jnp.float32(1.0 / N)
    mu = jnp.sum(s, axis=0, keepdims=True) * inv_n
    var = jnp.sum(s * s, axis=0, keepdims=True) * inv_n - mu * mu
    g = g_ref[...] * (s - mu) * lax.rsqrt(var + 1e-5) + be_ref[...]
    g = jnp.where(mask, jnp.maximum(g, 0.0), 0.0)
    h2 = jnp.dot(g, w2_ref[...], preferred_element_type=jnp.float32)
    hs2_ref[...] = h2 * dinv


def _tc3(acc_ref, hs2_ref, deg_ref, b2_ref, out_ref):
    dinv = _dinv_from(deg_ref)
    o = (acc_ref[0] + acc_ref[1] + hs2_ref[...]) * dinv + b2_ref[...]
    out_ref[...] = o[:N, :C]


def kernel(x, edge_index, W1, b1, gamma1, beta1, W2, b2):
    pad = jnp.full((EPAD - E,), N, dtype=jnp.int32)
    r_p = jnp.concatenate([edge_index[0], pad]).reshape(NW, CPW, CHUNK)
    c_p = jnp.concatenate([edge_index[1], pad]).reshape(NW, CPW, CHUNK)
    x_p = jnp.pad(x, ((0, NPAD - N), (0, 0)))
    w2_p = jnp.pad(W2, ((0, 0), (0, D2 - C)))
    b1r = b1.reshape(1, H)
    b2r = jnp.pad(b2, (0, D2 - C)).reshape(1, D2)
    g1r = gamma1.reshape(1, H)
    be1r = beta1.reshape(1, H)

    deg = _deg_kernel(c_p)

    hs1 = pl.pallas_call(
        _tc1,
        out_shape=jax.ShapeDtypeStruct((NPAD, H), jnp.float32),
    )(x_p, W1, deg)

    acc1 = _prop1(r_p, c_p, hs1)

    hs2 = pl.pallas_call(
        _tc2,
        out_shape=jax.ShapeDtypeStruct((NPAD, D2), jnp.float32),
    )(acc1, hs1, deg, g1r, be1r, b1r, w2_p)

    acc2 = _prop2(r_p, c_p, hs2)

    out = pl.pallas_call(
        _tc3,
        out_shape=jax.ShapeDtypeStruct((N, C), jnp.float32),
    )(acc2, hs2, deg, b2r)

    return out
